# baseline (speedup 1.0000x reference)
import functools

import jax
import jax.numpy as jnp
from jax import lax
from jax.experimental import pallas as pl
from jax.experimental.pallas import tpu as pltpu
from jax.experimental.pallas import tpu_sc as plsc

BS = 4
NP = 20000
C = 128
M = 512
NPOS = 128
NTILES_B = 8
PER_TILE = NP // NTILES_B
PT_PAD = 2560
NCHUNK = PT_PAD // 16
ROWS_PER_TILE = M // NTILES_B

_BIG_I = 2 ** 30


def _fps_gather_body(xp_hbm, yp_hbm, zp_hbm, fp_hbm, feat_hbm,
                     outf_hbm, scx_hbm, scy_hbm, scz_hbm,
                     x_v, y_v, z_v, d_v, idx_v, gidx_v,
                     sx_v, sy_v, sz_v, pub_v, all_v, fp_v, rows_v,
                     shared, sem):
    c = lax.axis_index("c")
    s = lax.axis_index("s")
    lane = lax.iota(jnp.int32, 16)
    zero16 = jnp.zeros((16,), jnp.int32)
    lane0 = lane == 0
    bslot = s // NTILES_B
    t = s - bslot * NTILES_B
    b = c * 2 + bslot
    tile_base = t * PER_TILE

    pltpu.sync_copy(xp_hbm.at[b, t], x_v)
    pltpu.sync_copy(yp_hbm.at[b, t], y_v)
    pltpu.sync_copy(zp_hbm.at[b, t], z_v)
    pltpu.sync_copy(fp_hbm.at[b], fp_v)

    def initd(j, carry):
        valid = (j * 16 + lane) < PER_TILE
        d_v[pl.ds(j * 16, 16)] = jnp.where(valid, jnp.float32(1e10),
                                           jnp.float32(-1.0))
        return carry
    lax.fori_loop(0, NCHUNK, initd, 0, unroll=4)

    fpv = fp_v[...]
    lx0 = jnp.max(jnp.where(lane == 0, fpv, jnp.float32(-1.0)))
    ly0 = jnp.max(jnp.where(lane == 1, fpv, jnp.float32(-1.0)))
    lz0 = jnp.max(jnp.where(lane == 2, fpv, jnp.float32(-1.0)))
    plsc.store_scatter(idx_v, [zero16], zero16, mask=lane0)

    @pl.when(t == 0)
    def _():
        plsc.store_scatter(sx_v, [zero16], jnp.broadcast_to(lx0, (16,)),
                           mask=lane0)
        plsc.store_scatter(sy_v, [zero16], jnp.broadcast_to(ly0, (16,)),
                           mask=lane0)
        plsc.store_scatter(sz_v, [zero16], jnp.broadcast_to(lz0, (16,)),
                           mask=lane0)

    def it_body(i, carry):
        lx, ly, lz = carry

        def quad(jj, st):
            st = list(st)
            for k in range(4):
                j = jj * 4 + k
                sl = pl.ds(j * 16, 16)
                dx = x_v[sl] - lx
                dy = y_v[sl] - ly
                dz = z_v[sl] - lz
                d = dx * dx + dy * dy + dz * dz
                dn = jnp.minimum(d_v[sl], d)
                d_v[sl] = dn
                gi = tile_base + j * 16 + lane
                bv, bi = st[2 * k], st[2 * k + 1]
                upd = dn > bv
                st[2 * k + 1] = jnp.where(upd, gi, bi)
                st[2 * k] = jnp.where(upd, dn, bv)
            return tuple(st)

        neg = jnp.full((16,), -3e38, jnp.float32)
        st = lax.fori_loop(
            0, NCHUNK // 4, quad,
            (neg, zero16, neg, zero16, neg, zero16, neg, zero16), unroll=2)

        def comb(av, ai, bv_, bi_):
            m = jnp.maximum(av, bv_)
            ca = jnp.where(av == m, ai, _BIG_I)
            cb = jnp.where(bv_ == m, bi_, _BIG_I)
            return m, jnp.minimum(ca, cb)

        v01, i01 = comb(st[0], st[1], st[2], st[3])
        v23, i23 = comb(st[4], st[5], st[6], st[7])
        bv, bi = comb(v01, i01, v23, i23)

        mt = jnp.max(bv)
        cand = jnp.where(bv == mt, bi, _BIG_I)
        it = jnp.min(cand)
        loc = it - tile_base
        loc16 = jnp.broadcast_to(loc, (16,))
        wx = plsc.load_gather(x_v, [loc16])
        wy = plsc.load_gather(y_v, [loc16])
        wz = plsc.load_gather(z_v, [loc16])

        it_f = plsc.bitcast(jnp.broadcast_to(it, (16,)), jnp.float32)
        pub = jnp.where(lane == 0, jnp.broadcast_to(mt, (16,)),
                        jnp.float32(0.0))
        pub = jnp.where(lane == 1, it_f, pub)
        pub = jnp.where(lane == 2, wx, pub)
        pub = jnp.where(lane == 3, wy, pub)
        pub = jnp.where(lane == 4, wz, pub)
        pub_v[...] = pub
        p = jnp.bitwise_and(i, 1)
        pltpu.sync_copy(pub_v, shared.at[pl.ds(p * 256 + s * 16, 16)])
        plsc.subcore_barrier()
        pltpu.sync_copy(shared.at[pl.ds(p * 256 + bslot * 128, 128)], all_v)

        row = jnp.bitwise_and(lane, 7) * 16
        vals = plsc.load_gather(all_v, [row])
        gidxs = plsc.bitcast(
            plsc.load_gather(all_v, [row + 1]), jnp.int32)
        mg = jnp.max(vals)
        cand2 = jnp.where(vals == mg, gidxs, _BIG_I)
        g = jnp.min(cand2)
        wmask = cand2 == g
        wxs = plsc.load_gather(all_v, [row + 2])
        wys = plsc.load_gather(all_v, [row + 3])
        wzs = plsc.load_gather(all_v, [row + 4])
        lxn = jnp.max(jnp.where(wmask, wxs, jnp.float32(-1.0)))
        lyn = jnp.max(jnp.where(wmask, wys, jnp.float32(-1.0)))
        lzn = jnp.max(jnp.where(wmask, wzs, jnp.float32(-1.0)))

        ipos = jnp.broadcast_to(i, (16,))
        plsc.store_scatter(idx_v, [ipos], jnp.broadcast_to(g, (16,)),
                           mask=lane0)

        @pl.when(t == 0)
        def _():
            plsc.store_scatter(sx_v, [ipos],
                               jnp.broadcast_to(lxn, (16,)), mask=lane0)
            plsc.store_scatter(sy_v, [ipos],
                               jnp.broadcast_to(lyn, (16,)), mask=lane0)
            plsc.store_scatter(sz_v, [ipos],
                               jnp.broadcast_to(lzn, (16,)), mask=lane0)

        return lxn, lyn, lzn

    lax.fori_loop(1, M, it_body, (lx0, ly0, lz0))

    @pl.when(t == 0)
    def _():
        pltpu.sync_copy(sx_v, scx_hbm.at[b])
        pltpu.sync_copy(sy_v, scy_hbm.at[b])
        pltpu.sync_copy(sz_v, scz_hbm.at[b])

    base = t * ROWS_PER_TILE
    for j in range(ROWS_PER_TILE // 16):
        gidx_v[pl.ds(j * 16, 16)] = idx_v[pl.ds(base + j * 16, 16)] + b * NP
    pltpu.async_copy(feat_hbm.at[gidx_v], rows_v, sem).wait()
    pltpu.sync_copy(rows_v, outf_hbm.at[b, pl.ds(base, ROWS_PER_TILE)])


@functools.partial(jax.jit, static_argnames=())
def _fps_gather(xpad, ypad, zpad, firstpt, features):
    mesh = plsc.VectorSubcoreMesh(core_axis_name="c", subcore_axis_name="s")
    fn = pl.kernel(
        _fps_gather_body,
        out_type=(
            jax.ShapeDtypeStruct((BS, M, C), jnp.float32),
            jax.ShapeDtypeStruct((BS, M), jnp.float32),
            jax.ShapeDtypeStruct((BS, M), jnp.float32),
            jax.ShapeDtypeStruct((BS, M), jnp.float32),
        ),
        mesh=mesh,
        scratch_types=[
            pltpu.VMEM((PT_PAD,), jnp.float32),
            pltpu.VMEM((PT_PAD,), jnp.float32),
            pltpu.VMEM((PT_PAD,), jnp.float32),
            pltpu.VMEM((PT_PAD,), jnp.float32),
            pltpu.VMEM((M,), jnp.int32),
            pltpu.VMEM((ROWS_PER_TILE,), jnp.int32),
            pltpu.VMEM((M,), jnp.float32),
            pltpu.VMEM((M,), jnp.float32),
            pltpu.VMEM((M,), jnp.float32),
            pltpu.VMEM((16,), jnp.float32),
            pltpu.VMEM((128,), jnp.float32),
            pltpu.VMEM((16,), jnp.float32),
            pltpu.VMEM((ROWS_PER_TILE, C), jnp.float32),
            pltpu.VMEM_SHARED((512,), jnp.float32),
            pltpu.SemaphoreType.DMA,
        ],
        compiler_params=pltpu.CompilerParams(needs_layout_passes=False),
    )
    return fn(xpad, ypad, zpad, firstpt, features)


def _pos_body(scx_ref, scy_ref, scz_ref, dimt_ref, w_ref, b_ref, out_ref):
    col = lax.broadcasted_iota(jnp.int32, (BS * M, NPOS), 1)
    even = (col % 2) == 0
    dimt = dimt_ref[...]
    embs = []
    for ref in (scx_ref, scy_ref, scz_ref):
        p = ref[...] / dimt
        embs.append(jnp.where(even, jnp.sin(p), jnp.cos(p)))
    e = jnp.concatenate(embs, axis=1)
    out_ref[...] = (
        jnp.dot(e, w_ref[...], preferred_element_type=jnp.float32)
        + b_ref[...]
    )


def _pos_embed(scx, scy, scz, dim_t, W_pos, b_pos):
    return pl.pallas_call(
        _pos_body,
        out_shape=jax.ShapeDtypeStruct((BS * M, NPOS), jnp.float32),
    )(scx, scy, scz, dim_t, W_pos, b_pos)


def kernel(features, batch_idx, coords, W_pos, b_pos):
    del batch_idx
    coords_b = coords.reshape(BS, NP, 3).astype(jnp.float32)
    ct = coords_b.transpose(0, 2, 1).reshape(BS, 3, NTILES_B, PER_TILE)
    ct = jnp.pad(ct, ((0, 0), (0, 0), (0, 0), (0, PT_PAD - PER_TILE)))
    xpad, ypad, zpad = ct[:, 0], ct[:, 1], ct[:, 2]
    firstpt = jnp.pad(coords_b[:, 0, :], ((0, 0), (0, 13)))

    outf, scx, scy, scz = _fps_gather(xpad, ypad, zpad, firstpt, features)

    dim_t = jnp.power(
        jnp.float32(10000.0),
        2.0 * (jnp.arange(NPOS, dtype=jnp.float32) // 2) / NPOS)
    scx2 = scx.reshape(BS * M, 1)
    scy2 = scy.reshape(BS * M, 1)
    scz2 = scz.reshape(BS * M, 1)
    pos = _pos_embed(scx2, scy2, scz2, dim_t.reshape(1, NPOS), W_pos,
                     b_pos.reshape(1, NPOS)).reshape(BS, M, NPOS)

    mask = jnp.zeros((BS, 1, M), dtype=bool)
    return outf, pos, mask


# speedup vs baseline: 2.3497x; 2.3497x over previous
import functools

import jax
import jax.numpy as jnp
from jax import lax
from jax.experimental import pallas as pl
from jax.experimental.pallas import tpu as pltpu
from jax.experimental.pallas import tpu_sc as plsc

BS = 4
NP = 20000
C = 128
M = 512
NPOS = 128
NTILES_B = 8
PER_TILE = NP // NTILES_B
PT_PAD = 2560
NCHUNK = PT_PAD // 16
ROWS_PER_TILE = M // NTILES_B

_BIG_I = 2 ** 30


def _fps_gather_body(xp_hbm, yp_hbm, zp_hbm, fp_hbm, feat_hbm,
                     outf_hbm, scx_hbm, scy_hbm, scz_hbm,
                     x_v, y_v, z_v, d_v, idx_v, gidx_v,
                     sx_v, sy_v, sz_v, pub_v, all_v, fp_v, rows_v,
                     shared, sem):
    c = lax.axis_index("c")
    s = lax.axis_index("s")
    lane = lax.iota(jnp.int32, 16)
    zero16 = jnp.zeros((16,), jnp.int32)
    lane0 = lane == 0
    bslot = s // NTILES_B
    t = s - bslot * NTILES_B
    b = c * 2 + bslot
    tile_base = t * PER_TILE

    pltpu.sync_copy(xp_hbm.at[b, t], x_v)
    pltpu.sync_copy(yp_hbm.at[b, t], y_v)
    pltpu.sync_copy(zp_hbm.at[b, t], z_v)
    pltpu.sync_copy(fp_hbm.at[b], fp_v)

    def initd(j, carry):
        valid = (j * 16 + lane) < PER_TILE
        d_v[pl.ds(j * 16, 16)] = jnp.where(valid, jnp.float32(1e10),
                                           jnp.float32(-1.0))
        return carry
    lax.fori_loop(0, NCHUNK, initd, 0, unroll=4)

    fpv = fp_v[...]
    lx0 = jnp.max(jnp.where(lane == 0, fpv, jnp.float32(-1.0)))
    ly0 = jnp.max(jnp.where(lane == 1, fpv, jnp.float32(-1.0)))
    lz0 = jnp.max(jnp.where(lane == 2, fpv, jnp.float32(-1.0)))
    plsc.store_scatter(idx_v, [zero16], zero16, mask=lane0)

    @pl.when(t == 0)
    def _():
        plsc.store_scatter(sx_v, [zero16], jnp.broadcast_to(lx0, (16,)),
                           mask=lane0)
        plsc.store_scatter(sy_v, [zero16], jnp.broadcast_to(ly0, (16,)),
                           mask=lane0)
        plsc.store_scatter(sz_v, [zero16], jnp.broadcast_to(lz0, (16,)),
                           mask=lane0)

    def it_body(i, carry):
        lx, ly, lz = carry

        def quad(jj, st):
            st = list(st)
            for k in range(4):
                j = jj * 4 + k
                sl = pl.ds(j * 16, 16)
                dx = x_v[sl] - lx
                dy = y_v[sl] - ly
                dz = z_v[sl] - lz
                d = dx * dx + dy * dy + dz * dz
                dn = jnp.minimum(d_v[sl], d)
                d_v[sl] = dn
                gi = tile_base + j * 16 + lane
                bv, bi = st[2 * k], st[2 * k + 1]
                upd = dn > bv
                st[2 * k + 1] = jnp.where(upd, gi, bi)
                st[2 * k] = jnp.where(upd, dn, bv)
            return tuple(st)

        neg = jnp.full((16,), -3e38, jnp.float32)
        st = plsc.parallel_loop(
            0, NCHUNK // 4, carry=(neg, zero16, neg, zero16,
                                   neg, zero16, neg, zero16),
            unroll=2)(quad)

        def comb(av, ai, bv_, bi_):
            m = jnp.maximum(av, bv_)
            ca = jnp.where(av == m, ai, _BIG_I)
            cb = jnp.where(bv_ == m, bi_, _BIG_I)
            return m, jnp.minimum(ca, cb)

        v01, i01 = comb(st[0], st[1], st[2], st[3])
        v23, i23 = comb(st[4], st[5], st[6], st[7])
        bv, bi = comb(v01, i01, v23, i23)

        mt = jnp.max(bv)
        cand = jnp.where(bv == mt, bi, _BIG_I)
        it = jnp.min(cand)
        loc = it - tile_base
        loc16 = jnp.broadcast_to(loc, (16,))
        wx = plsc.load_gather(x_v, [loc16])
        wy = plsc.load_gather(y_v, [loc16])
        wz = plsc.load_gather(z_v, [loc16])

        it_f = plsc.bitcast(jnp.broadcast_to(it, (16,)), jnp.float32)
        pub = jnp.where(lane == 0, jnp.broadcast_to(mt, (16,)),
                        jnp.float32(0.0))
        pub = jnp.where(lane == 1, it_f, pub)
        pub = jnp.where(lane == 2, wx, pub)
        pub = jnp.where(lane == 3, wy, pub)
        pub = jnp.where(lane == 4, wz, pub)
        pub_v[...] = pub
        p = jnp.bitwise_and(i, 1)
        pltpu.sync_copy(pub_v, shared.at[pl.ds(p * 256 + s * 16, 16)])
        plsc.subcore_barrier()
        pltpu.sync_copy(shared.at[pl.ds(p * 256 + bslot * 128, 128)], all_v)

        row = jnp.bitwise_and(lane, 7) * 16
        vals = plsc.load_gather(all_v, [row])
        gidxs = plsc.bitcast(
            plsc.load_gather(all_v, [row + 1]), jnp.int32)
        mg = jnp.max(vals)
        cand2 = jnp.where(vals == mg, gidxs, _BIG_I)
        g = jnp.min(cand2)
        wmask = cand2 == g
        wxs = plsc.load_gather(all_v, [row + 2])
        wys = plsc.load_gather(all_v, [row + 3])
        wzs = plsc.load_gather(all_v, [row + 4])
        lxn = jnp.max(jnp.where(wmask, wxs, jnp.float32(-1.0)))
        lyn = jnp.max(jnp.where(wmask, wys, jnp.float32(-1.0)))
        lzn = jnp.max(jnp.where(wmask, wzs, jnp.float32(-1.0)))

        ipos = jnp.broadcast_to(i, (16,))
        plsc.store_scatter(idx_v, [ipos], jnp.broadcast_to(g, (16,)),
                           mask=lane0)

        @pl.when(t == 0)
        def _():
            plsc.store_scatter(sx_v, [ipos],
                               jnp.broadcast_to(lxn, (16,)), mask=lane0)
            plsc.store_scatter(sy_v, [ipos],
                               jnp.broadcast_to(lyn, (16,)), mask=lane0)
            plsc.store_scatter(sz_v, [ipos],
                               jnp.broadcast_to(lzn, (16,)), mask=lane0)

        return lxn, lyn, lzn

    lax.fori_loop(1, M, it_body, (lx0, ly0, lz0))

    @pl.when(t == 0)
    def _():
        pltpu.sync_copy(sx_v, scx_hbm.at[b])
        pltpu.sync_copy(sy_v, scy_hbm.at[b])
        pltpu.sync_copy(sz_v, scz_hbm.at[b])

    base = t * ROWS_PER_TILE
    for j in range(ROWS_PER_TILE // 16):
        gidx_v[pl.ds(j * 16, 16)] = idx_v[pl.ds(base + j * 16, 16)] + b * NP
    pltpu.async_copy(feat_hbm.at[gidx_v], rows_v, sem).wait()
    pltpu.sync_copy(rows_v, outf_hbm.at[b, pl.ds(base, ROWS_PER_TILE)])


@functools.partial(jax.jit, static_argnames=())
def _fps_gather(xpad, ypad, zpad, firstpt, features):
    mesh = plsc.VectorSubcoreMesh(core_axis_name="c", subcore_axis_name="s")
    fn = pl.kernel(
        _fps_gather_body,
        out_type=(
            jax.ShapeDtypeStruct((BS, M, C), jnp.float32),
            jax.ShapeDtypeStruct((BS, M), jnp.float32),
            jax.ShapeDtypeStruct((BS, M), jnp.float32),
            jax.ShapeDtypeStruct((BS, M), jnp.float32),
        ),
        mesh=mesh,
        scratch_types=[
            pltpu.VMEM((PT_PAD,), jnp.float32),
            pltpu.VMEM((PT_PAD,), jnp.float32),
            pltpu.VMEM((PT_PAD,), jnp.float32),
            pltpu.VMEM((PT_PAD,), jnp.float32),
            pltpu.VMEM((M,), jnp.int32),
            pltpu.VMEM((ROWS_PER_TILE,), jnp.int32),
            pltpu.VMEM((M,), jnp.float32),
            pltpu.VMEM((M,), jnp.float32),
            pltpu.VMEM((M,), jnp.float32),
            pltpu.VMEM((16,), jnp.float32),
            pltpu.VMEM((128,), jnp.float32),
            pltpu.VMEM((16,), jnp.float32),
            pltpu.VMEM((ROWS_PER_TILE, C), jnp.float32),
            pltpu.VMEM_SHARED((512,), jnp.float32),
            pltpu.SemaphoreType.DMA,
        ],
        compiler_params=pltpu.CompilerParams(needs_layout_passes=False),
    )
    return fn(xpad, ypad, zpad, firstpt, features)


def _pos_body(scx_ref, scy_ref, scz_ref, dimt_ref, w_ref, b_ref, out_ref):
    col = lax.broadcasted_iota(jnp.int32, (BS * M, NPOS), 1)
    even = (col % 2) == 0
    dimt = dimt_ref[...]
    embs = []
    for ref in (scx_ref, scy_ref, scz_ref):
        p = ref[...] / dimt
        embs.append(jnp.where(even, jnp.sin(p), jnp.cos(p)))
    e = jnp.concatenate(embs, axis=1)
    out_ref[...] = (
        jnp.dot(e, w_ref[...], preferred_element_type=jnp.float32)
        + b_ref[...]
    )


def _pos_embed(scx, scy, scz, dim_t, W_pos, b_pos):
    return pl.pallas_call(
        _pos_body,
        out_shape=jax.ShapeDtypeStruct((BS * M, NPOS), jnp.float32),
    )(scx, scy, scz, dim_t, W_pos, b_pos)


def kernel(features, batch_idx, coords, W_pos, b_pos):
    del batch_idx
    coords_b = coords.reshape(BS, NP, 3).astype(jnp.float32)
    ct = coords_b.transpose(0, 2, 1).reshape(BS, 3, NTILES_B, PER_TILE)
    ct = jnp.pad(ct, ((0, 0), (0, 0), (0, 0), (0, PT_PAD - PER_TILE)))
    xpad, ypad, zpad = ct[:, 0], ct[:, 1], ct[:, 2]
    firstpt = jnp.pad(coords_b[:, 0, :], ((0, 0), (0, 13)))

    outf, scx, scy, scz = _fps_gather(xpad, ypad, zpad, firstpt, features)

    dim_t = jnp.power(
        jnp.float32(10000.0),
        2.0 * (jnp.arange(NPOS, dtype=jnp.float32) // 2) / NPOS)
    scx2 = scx.reshape(BS * M, 1)
    scy2 = scy.reshape(BS * M, 1)
    scz2 = scz.reshape(BS * M, 1)
    pos = _pos_embed(scx2, scy2, scz2, dim_t.reshape(1, NPOS), W_pos,
                     b_pos.reshape(1, NPOS)).reshape(BS, M, NPOS)

    mask = jnp.zeros((BS, 1, M), dtype=bool)
    return outf, pos, mask


# speedup vs baseline: 2.3736x; 1.0102x over previous
import functools

import jax
import jax.numpy as jnp
from jax import lax
from jax.experimental import pallas as pl
from jax.experimental.pallas import tpu as pltpu
from jax.experimental.pallas import tpu_sc as plsc

BS = 4
NP = 20000
C = 128
M = 512
NPOS = 128
NTILES_B = 8
PER_TILE = NP // NTILES_B
PT_PAD = 2560
NCHUNK = PT_PAD // 16
ROWS_PER_TILE = M // NTILES_B

_BIG_I = 2 ** 30


def _fps_gather_body(xp_hbm, yp_hbm, zp_hbm, fp_hbm, feat_hbm,
                     outf_hbm, scx_hbm, scy_hbm, scz_hbm,
                     x_v, y_v, z_v, d_v, idx_v, gidx_v,
                     sx_v, sy_v, sz_v, pub_v, all_v, fp_v, rows_v,
                     shared, sem):
    c = lax.axis_index("c")
    s = lax.axis_index("s")
    lane = lax.iota(jnp.int32, 16)
    zero16 = jnp.zeros((16,), jnp.int32)
    lane0 = lane == 0
    bslot = s // NTILES_B
    t = s - bslot * NTILES_B
    b = c * 2 + bslot
    tile_base = t * PER_TILE

    pltpu.sync_copy(xp_hbm.at[b, t], x_v)
    pltpu.sync_copy(yp_hbm.at[b, t], y_v)
    pltpu.sync_copy(zp_hbm.at[b, t], z_v)
    pltpu.sync_copy(fp_hbm.at[b], fp_v)

    def initd(j, carry):
        valid = (j * 16 + lane) < PER_TILE
        d_v[pl.ds(j * 16, 16)] = jnp.where(valid, jnp.float32(1e10),
                                           jnp.float32(-1.0))
        return carry
    lax.fori_loop(0, NCHUNK, initd, 0, unroll=4)

    fpv = fp_v[...]
    lx0 = jnp.max(jnp.where(lane == 0, fpv, jnp.float32(-1.0)))
    ly0 = jnp.max(jnp.where(lane == 1, fpv, jnp.float32(-1.0)))
    lz0 = jnp.max(jnp.where(lane == 2, fpv, jnp.float32(-1.0)))
    plsc.store_scatter(idx_v, [zero16], zero16, mask=lane0)

    @pl.when(t == 0)
    def _():
        plsc.store_scatter(sx_v, [zero16], jnp.broadcast_to(lx0, (16,)),
                           mask=lane0)
        plsc.store_scatter(sy_v, [zero16], jnp.broadcast_to(ly0, (16,)),
                           mask=lane0)
        plsc.store_scatter(sz_v, [zero16], jnp.broadcast_to(lz0, (16,)),
                           mask=lane0)

    def it_body(i, carry):
        lx, ly, lz = carry

        def quad(jj, st):
            st = list(st)
            for k in range(4):
                j = jj * 4 + k
                sl = pl.ds(j * 16, 16)
                dx = x_v[sl] - lx
                dy = y_v[sl] - ly
                dz = z_v[sl] - lz
                d = dx * dx + dy * dy + dz * dz
                dn = jnp.minimum(d_v[sl], d)
                d_v[sl] = dn
                gi = tile_base + j * 16 + lane
                bv, bi = st[2 * k], st[2 * k + 1]
                upd = dn > bv
                st[2 * k + 1] = jnp.where(upd, gi, bi)
                st[2 * k] = jnp.where(upd, dn, bv)
            return tuple(st)

        neg = jnp.full((16,), -3e38, jnp.float32)
        st = plsc.parallel_loop(
            0, NCHUNK // 4, carry=(neg, zero16, neg, zero16,
                                   neg, zero16, neg, zero16),
            unroll=4)(quad)

        def comb(av, ai, bv_, bi_):
            m = jnp.maximum(av, bv_)
            ca = jnp.where(av == m, ai, _BIG_I)
            cb = jnp.where(bv_ == m, bi_, _BIG_I)
            return m, jnp.minimum(ca, cb)

        v01, i01 = comb(st[0], st[1], st[2], st[3])
        v23, i23 = comb(st[4], st[5], st[6], st[7])
        bv, bi = comb(v01, i01, v23, i23)

        mt = jnp.max(bv)
        cand = jnp.where(bv == mt, bi, _BIG_I)
        it = jnp.min(cand)
        loc = it - tile_base
        loc16 = jnp.broadcast_to(loc, (16,))
        wx = plsc.load_gather(x_v, [loc16])
        wy = plsc.load_gather(y_v, [loc16])
        wz = plsc.load_gather(z_v, [loc16])

        it_f = plsc.bitcast(jnp.broadcast_to(it, (16,)), jnp.float32)
        pub = jnp.where(lane == 0, jnp.broadcast_to(mt, (16,)),
                        jnp.float32(0.0))
        pub = jnp.where(lane == 1, it_f, pub)
        pub = jnp.where(lane == 2, wx, pub)
        pub = jnp.where(lane == 3, wy, pub)
        pub = jnp.where(lane == 4, wz, pub)
        pub_v[...] = pub
        p = jnp.bitwise_and(i, 1)
        pltpu.sync_copy(pub_v, shared.at[pl.ds(p * 256 + s * 16, 16)])
        plsc.subcore_barrier()
        pltpu.sync_copy(shared.at[pl.ds(p * 256 + bslot * 128, 128)], all_v)

        row = jnp.bitwise_and(lane, 7) * 16
        vals = plsc.load_gather(all_v, [row])
        gidxs = plsc.bitcast(
            plsc.load_gather(all_v, [row + 1]), jnp.int32)
        mg = jnp.max(vals)
        cand2 = jnp.where(vals == mg, gidxs, _BIG_I)
        g = jnp.min(cand2)
        wmask = cand2 == g
        wxs = plsc.load_gather(all_v, [row + 2])
        wys = plsc.load_gather(all_v, [row + 3])
        wzs = plsc.load_gather(all_v, [row + 4])
        lxn = jnp.max(jnp.where(wmask, wxs, jnp.float32(-1.0)))
        lyn = jnp.max(jnp.where(wmask, wys, jnp.float32(-1.0)))
        lzn = jnp.max(jnp.where(wmask, wzs, jnp.float32(-1.0)))

        ipos = jnp.broadcast_to(i, (16,))
        plsc.store_scatter(idx_v, [ipos], jnp.broadcast_to(g, (16,)),
                           mask=lane0)

        @pl.when(t == 0)
        def _():
            plsc.store_scatter(sx_v, [ipos],
                               jnp.broadcast_to(lxn, (16,)), mask=lane0)
            plsc.store_scatter(sy_v, [ipos],
                               jnp.broadcast_to(lyn, (16,)), mask=lane0)
            plsc.store_scatter(sz_v, [ipos],
                               jnp.broadcast_to(lzn, (16,)), mask=lane0)

        return lxn, lyn, lzn

    lax.fori_loop(1, M, it_body, (lx0, ly0, lz0))

    @pl.when(t == 0)
    def _():
        pltpu.sync_copy(sx_v, scx_hbm.at[b])
        pltpu.sync_copy(sy_v, scy_hbm.at[b])
        pltpu.sync_copy(sz_v, scz_hbm.at[b])

    base = t * ROWS_PER_TILE
    for j in range(ROWS_PER_TILE // 16):
        gidx_v[pl.ds(j * 16, 16)] = idx_v[pl.ds(base + j * 16, 16)] + b * NP
    pltpu.async_copy(feat_hbm.at[gidx_v], rows_v, sem).wait()
    pltpu.sync_copy(rows_v, outf_hbm.at[b, pl.ds(base, ROWS_PER_TILE)])


@functools.partial(jax.jit, static_argnames=())
def _fps_gather(xpad, ypad, zpad, firstpt, features):
    mesh = plsc.VectorSubcoreMesh(core_axis_name="c", subcore_axis_name="s")
    fn = pl.kernel(
        _fps_gather_body,
        out_type=(
            jax.ShapeDtypeStruct((BS, M, C), jnp.float32),
            jax.ShapeDtypeStruct((BS, M), jnp.float32),
            jax.ShapeDtypeStruct((BS, M), jnp.float32),
            jax.ShapeDtypeStruct((BS, M), jnp.float32),
        ),
        mesh=mesh,
        scratch_types=[
            pltpu.VMEM((PT_PAD,), jnp.float32),
            pltpu.VMEM((PT_PAD,), jnp.float32),
            pltpu.VMEM((PT_PAD,), jnp.float32),
            pltpu.VMEM((PT_PAD,), jnp.float32),
            pltpu.VMEM((M,), jnp.int32),
            pltpu.VMEM((ROWS_PER_TILE,), jnp.int32),
            pltpu.VMEM((M,), jnp.float32),
            pltpu.VMEM((M,), jnp.float32),
            pltpu.VMEM((M,), jnp.float32),
            pltpu.VMEM((16,), jnp.float32),
            pltpu.VMEM((128,), jnp.float32),
            pltpu.VMEM((16,), jnp.float32),
            pltpu.VMEM((ROWS_PER_TILE, C), jnp.float32),
            pltpu.VMEM_SHARED((512,), jnp.float32),
            pltpu.SemaphoreType.DMA,
        ],
        compiler_params=pltpu.CompilerParams(
            needs_layout_passes=False, disable_bounds_checks=True),
    )
    return fn(xpad, ypad, zpad, firstpt, features)


def _pos_body(scx_ref, scy_ref, scz_ref, dimt_ref, w_ref, b_ref, out_ref):
    col = lax.broadcasted_iota(jnp.int32, (BS * M, NPOS), 1)
    even = (col % 2) == 0
    dimt = dimt_ref[...]
    embs = []
    for ref in (scx_ref, scy_ref, scz_ref):
        p = ref[...] / dimt
        embs.append(jnp.where(even, jnp.sin(p), jnp.cos(p)))
    e = jnp.concatenate(embs, axis=1)
    out_ref[...] = (
        jnp.dot(e, w_ref[...], preferred_element_type=jnp.float32)
        + b_ref[...]
    )


def _pos_embed(scx, scy, scz, dim_t, W_pos, b_pos):
    return pl.pallas_call(
        _pos_body,
        out_shape=jax.ShapeDtypeStruct((BS * M, NPOS), jnp.float32),
    )(scx, scy, scz, dim_t, W_pos, b_pos)


def kernel(features, batch_idx, coords, W_pos, b_pos):
    del batch_idx
    coords_b = coords.reshape(BS, NP, 3).astype(jnp.float32)
    ct = coords_b.transpose(0, 2, 1).reshape(BS, 3, NTILES_B, PER_TILE)
    ct = jnp.pad(ct, ((0, 0), (0, 0), (0, 0), (0, PT_PAD - PER_TILE)))
    xpad, ypad, zpad = ct[:, 0], ct[:, 1], ct[:, 2]
    firstpt = jnp.pad(coords_b[:, 0, :], ((0, 0), (0, 13)))

    outf, scx, scy, scz = _fps_gather(xpad, ypad, zpad, firstpt, features)

    dim_t = jnp.power(
        jnp.float32(10000.0),
        2.0 * (jnp.arange(NPOS, dtype=jnp.float32) // 2) / NPOS)
    scx2 = scx.reshape(BS * M, 1)
    scy2 = scy.reshape(BS * M, 1)
    scz2 = scz.reshape(BS * M, 1)
    pos = _pos_embed(scx2, scy2, scz2, dim_t.reshape(1, NPOS), W_pos,
                     b_pos.reshape(1, NPOS)).reshape(BS, M, NPOS)

    mask = jnp.zeros((BS, 1, M), dtype=bool)
    return outf, pos, mask
